# baseline (speedup 1.0000x reference)
import functools

import jax
import jax.numpy as jnp
from jax import lax
from jax.experimental import pallas as pl
from jax.experimental.pallas import tpu as pltpu
from jax.experimental.pallas import tpu_sc as plsc

TEMPERATURE = 0.07
TOPK = 16
OFFSET_SCALE = 0.5
SPATIAL = (16, 16, 16)
B, N, C, H = 2, 4096, 128, 128

BLK_SIM = 256
BLK_REF = 256
GCHUNK = 128
TABW = 256


def _norm_proj_body(t_ref, w_ref, tn_ref, proj_ref):
    t = t_ref[0]
    norm = jnp.sqrt(jnp.sum(t * t, axis=-1, keepdims=True)) + 1e-8
    tn_ref[0] = t / norm
    proj_ref[0] = jnp.dot(t, w_ref[...], preferred_element_type=jnp.float32)


def _simtopk_body(s_ref, tn_ref, vals_ref, idx_ref):
    b = pl.program_id(0)
    s = s_ref[0]
    sn = s / (jnp.sqrt(jnp.sum(s * s, axis=-1, keepdims=True)) + 1e-8)
    tn = tn_ref[0]
    sim = lax.dot_general(sn, tn, (((1,), (1,)), ((), ())),
                          preferred_element_type=jnp.float32) / TEMPERATURE
    r_iota = lax.broadcasted_iota(jnp.int32, (N, 8), 0)
    c_iota = lax.broadcasted_iota(jnp.int32, (N, 8), 1)
    iota_col = jnp.where(
        c_iota == 0, r_iota // 64,
        jnp.where(c_iota == 1, r_iota % 64,
                  jnp.where(c_iota == 2, 1, 0))).astype(jnp.float32)
    iota32 = lax.broadcasted_iota(jnp.int32, (BLK_SIM, N), 1)
    kiota = lax.broadcasted_iota(jnp.int32, (BLK_SIM, TOPK), 1)
    vacc = jnp.zeros((BLK_SIM, TOPK), jnp.float32)
    iacc = jnp.zeros((BLK_SIM, TOPK), jnp.int32)
    for k in range(TOPK):
        m = jnp.max(sim, axis=1)
        oh = jnp.where(sim == m[:, None], 1.0, 0.0)
        ex = lax.dot_general(oh, iota_col, (((1,), (0,)), ((), ())),
                             preferred_element_type=jnp.float32)
        mi_cheap = jnp.clip(ex[:, 0].astype(jnp.int32) * 64
                            + ex[:, 1].astype(jnp.int32), 0, N - 1)
        simk, mk = sim, m
        mi = lax.cond(
            jnp.max(ex[:, 2]) > 1.5,
            lambda: jnp.min(jnp.where(simk == mk[:, None], iota32, N), axis=1),
            lambda: mi_cheap)
        vacc = jnp.where(kiota == k, m[:, None], vacc)
        iacc = jnp.where(kiota == k, mi[:, None], iacc)
        if k < TOPK - 1:
            sim = jnp.where(iota32 == mi[:, None], -1e30, sim)
    vals_ref[0] = vacc
    idx_ref[0] = iacc + b * N


def _refine_body(g_ref, v_ref, sd_ref, ss_ref,
                 w1s_ref, w1c_ref, b1_ref, w2_ref, b2_ref, wlo_ref, blo_ref,
                 exp_ref, disp_ref, probs_ref, conf_ref, marg_ref, ent_ref):
    gall = g_ref[0]
    g = gall[..., 0:H]
    v3 = v_ref[0]
    sd = sd_ref[0]
    ss = ss_ref[0]
    blk = g.shape[0]

    cand_canon = gall[..., H:H + 3]
    cand_pos = gall[..., H + 3:H + 6]
    cu3 = gall[..., H + 6:H + 7]
    cm3 = gall[..., H + 7:H + 8]
    delta = cand_canon - ss[:, None, 0:3]
    offs = cand_pos - ss[:, None, 3:6]
    dist = jnp.sqrt(jnp.sum(delta * delta, axis=-1, keepdims=True))
    su3 = jnp.broadcast_to(ss[:, None, 6:7], (blk, TOPK, 1))
    sm3 = jnp.broadcast_to(ss[:, None, 7:8], (blk, TOPK, 1))
    scal = jnp.concatenate([v3, delta, offs, dist, su3, cu3, sm3, cm3], -1)

    srcp = jnp.dot(sd, w1s_ref[...], preferred_element_type=jnp.float32)
    scalp = lax.dot_general(scal, w1c_ref[...], (((2,), (0,)), ((), ())),
                            preferred_element_type=jnp.float32)
    h = jax.nn.gelu(g + srcp[:, None, :] + scalp + b1_ref[...][None])
    h = jax.nn.gelu(
        lax.dot_general(h, w2_ref[...], (((2,), (0,)), ((), ())),
                        preferred_element_type=jnp.float32) + b2_ref[...][None])
    out4 = lax.dot_general(h, wlo_ref[...], (((2,), (0,)), ((), ())),
                           preferred_element_type=jnp.float32) + blo_ref[...][None]

    logits = v3[..., 0] + out4[..., 0]
    offset = jnp.tanh(out4[..., 1:4]) * OFFSET_SCALE
    m = jnp.max(logits, axis=-1, keepdims=True)
    e = jnp.exp(logits - m)
    p = e / jnp.sum(e, axis=-1, keepdims=True)

    refined = cand_pos + offset
    expected = jnp.sum(p[..., None] * refined, axis=1)
    disp = expected - ss[:, 3:6]

    kiota = lax.broadcasted_iota(jnp.int32, p.shape, 1)
    top1 = jnp.max(p, axis=-1, keepdims=True)
    idx1 = jnp.min(jnp.where(p == top1, kiota, TOPK), axis=-1, keepdims=True)
    masked = jnp.where(kiota == idx1, -1.0, p)
    top2 = jnp.max(masked, axis=-1, keepdims=True)
    margin = top1 - top2
    btm = jnp.sum(jnp.where(kiota == idx1, cm3[..., 0], 0.0), axis=-1,
                  keepdims=True)
    conf = top1 * jnp.sqrt(jnp.maximum(ss[:, 7:8] * btm, 1e-6))
    ent = -jnp.sum(p * jnp.log(p + 1e-12), axis=-1, keepdims=True)

    exp_ref[0] = expected
    disp_ref[0] = jnp.nan_to_num(disp, nan=0.0, posinf=0.0, neginf=0.0)
    probs_ref[0] = jnp.nan_to_num(p, nan=0.0, posinf=0.0, neginf=0.0)
    conf_ref[0] = conf
    marg_ref[0] = jnp.nan_to_num(margin, nan=0.0, posinf=0.0, neginf=0.0)
    ent_ref[0] = jnp.nan_to_num(ent, nan=0.0, posinf=0.0, neginf=0.0)


def _gather_sc(tab, idx2):
    rows = idx2.shape[0] * idx2.shape[1]
    info = plsc.get_sparse_core_info()
    nw = info.num_cores * info.num_subcores
    rows_w = rows // nw
    n_chunks = rows_w // GCHUNK
    idx_rows_w = rows_w // idx2.shape[1]
    mesh = plsc.VectorSubcoreMesh(core_axis_name="c", subcore_axis_name="s")

    @functools.partial(
        pl.kernel, mesh=mesh,
        out_type=jax.ShapeDtypeStruct((rows, TABW), jnp.float32),
        scratch_types=[pltpu.VMEM((idx_rows_w, 128), jnp.int32),
                       pltpu.VMEM((GCHUNK, TABW), jnp.float32),
                       pltpu.SemaphoreType.DMA],
    )
    def gk(tab_hbm, idx_hbm, out, idx_v, buf, sem):
        wid = lax.axis_index("s") * info.num_cores + lax.axis_index("c")
        pltpu.sync_copy(idx_hbm.at[pl.ds(wid * idx_rows_w, idx_rows_w)], idx_v)

        def body(c, carry):
            pltpu.async_copy(tab_hbm.at[idx_v.at[c]], buf, sem).wait()
            base = wid * rows_w + c * GCHUNK
            pltpu.sync_copy(buf, out.at[pl.ds(base, GCHUNK)])
            return carry

        lax.fori_loop(0, n_chunks, body, 0)

    return gk(tab, idx2)


def kernel(src_descriptor, tgt_descriptor, src_canonical, tgt_canonical,
           src_positions, tgt_positions, src_uncertainty, tgt_uncertainty,
           src_matchability, tgt_matchability,
           W1, b1, W2, b2, Wl, bl, Wo, bo):
    f32 = jnp.float32

    tn, tgt_proj = pl.pallas_call(
        _norm_proj_body,
        grid=(B,),
        in_specs=[pl.BlockSpec((1, N, C), lambda b: (b, 0, 0)),
                  pl.BlockSpec((C, H), lambda b: (0, 0))],
        out_specs=[pl.BlockSpec((1, N, C), lambda b: (b, 0, 0)),
                   pl.BlockSpec((1, N, H), lambda b: (b, 0, 0))],
        out_shape=[jax.ShapeDtypeStruct((B, N, C), f32),
                   jax.ShapeDtypeStruct((B, N, H), f32)],
    )(tgt_descriptor, W1[C:2 * C])

    vals, idxf = pl.pallas_call(
        _simtopk_body,
        grid=(B, N // BLK_SIM),
        in_specs=[pl.BlockSpec((1, BLK_SIM, C), lambda b, i: (b, i, 0)),
                  pl.BlockSpec((1, N, C), lambda b, i: (b, 0, 0))],
        out_specs=[pl.BlockSpec((1, BLK_SIM, TOPK), lambda b, i: (b, i, 0)),
                   pl.BlockSpec((1, BLK_SIM, TOPK), lambda b, i: (b, i, 0))],
        out_shape=[jax.ShapeDtypeStruct((B, N, TOPK), f32),
                   jax.ShapeDtypeStruct((B, N, TOPK), jnp.int32)],
    )(src_descriptor, tn)

    tab = jnp.concatenate(
        [tgt_proj, tgt_canonical, tgt_positions, tgt_uncertainty[..., None],
         tgt_matchability[..., None],
         jnp.zeros((B, N, TABW - H - 8), f32)], axis=-1)
    rows = B * N * TOPK
    ga = _gather_sc(tab.reshape(B * N, TABW),
                    idxf.reshape(rows // 128, 128))
    g4 = ga.reshape(B, N, TOPK, TABW)

    src_small = jnp.concatenate(
        [src_canonical, src_positions, src_uncertainty[..., None],
         src_matchability[..., None]], axis=-1)
    wlo = jnp.concatenate([Wl, Wo], axis=1)
    blo = jnp.concatenate([bl, bo])[None]

    outs = pl.pallas_call(
        _refine_body,
        grid=(B, N // BLK_REF),
        in_specs=[
            pl.BlockSpec((1, BLK_REF, TOPK, TABW), lambda b, i: (b, i, 0, 0)),
            pl.BlockSpec((1, BLK_REF, TOPK, 1), lambda b, i: (b, i, 0, 0)),
            pl.BlockSpec((1, BLK_REF, C), lambda b, i: (b, i, 0)),
            pl.BlockSpec((1, BLK_REF, 8), lambda b, i: (b, i, 0)),
            pl.BlockSpec((C, H), lambda b, i: (0, 0)),
            pl.BlockSpec((12, H), lambda b, i: (0, 0)),
            pl.BlockSpec((1, H), lambda b, i: (0, 0)),
            pl.BlockSpec((H, H), lambda b, i: (0, 0)),
            pl.BlockSpec((1, H), lambda b, i: (0, 0)),
            pl.BlockSpec((H, 4), lambda b, i: (0, 0)),
            pl.BlockSpec((1, 4), lambda b, i: (0, 0)),
        ],
        out_specs=[
            pl.BlockSpec((1, BLK_REF, 3), lambda b, i: (b, i, 0)),
            pl.BlockSpec((1, BLK_REF, 3), lambda b, i: (b, i, 0)),
            pl.BlockSpec((1, BLK_REF, TOPK), lambda b, i: (b, i, 0)),
            pl.BlockSpec((1, BLK_REF, 1), lambda b, i: (b, i, 0)),
            pl.BlockSpec((1, BLK_REF, 1), lambda b, i: (b, i, 0)),
            pl.BlockSpec((1, BLK_REF, 1), lambda b, i: (b, i, 0)),
        ],
        out_shape=[
            jax.ShapeDtypeStruct((B, N, 3), f32),
            jax.ShapeDtypeStruct((B, N, 3), f32),
            jax.ShapeDtypeStruct((B, N, TOPK), f32),
            jax.ShapeDtypeStruct((B, N, 1), f32),
            jax.ShapeDtypeStruct((B, N, 1), f32),
            jax.ShapeDtypeStruct((B, N, 1), f32),
        ],
    )(g4, vals[..., None], src_descriptor, src_small,
      W1[0:C], W1[2 * C:], b1[None], W2, b2[None], wlo, blo)

    expected, disp, probs, conf, margin, ent = outs
    raw_disp = jnp.transpose(disp, (0, 2, 1)).reshape(B, 3, *SPATIAL)
    confidence = conf.reshape(B, 1, *SPATIAL)
    margin = margin.reshape(B, 1, *SPATIAL)
    entropy = ent.reshape(B, 1, *SPATIAL)
    return (expected, raw_disp, probs, confidence, margin, entropy)


# speedup vs baseline: 1.4572x; 1.4572x over previous
import functools

import jax
import jax.numpy as jnp
from jax import lax
from jax.experimental import pallas as pl
from jax.experimental.pallas import tpu as pltpu
from jax.experimental.pallas import tpu_sc as plsc

TEMPERATURE = 0.07
TOPK = 16
OFFSET_SCALE = 0.5
SPATIAL = (16, 16, 16)
B, N, C, H = 2, 4096, 128, 128

BLK_SIM = 256
BLK_REF = 256
GCHUNK = 128
TABW = 256


def _norm_proj_body(t_ref, w_ref, s8_ref, wf_ref, tn_ref, proj_ref):
    t = t_ref[0]
    norm = jnp.sqrt(jnp.sum(t * t, axis=-1, keepdims=True)) + 1e-8
    tn_ref[0] = t / norm
    proj_ref[0] = (
        jnp.dot(t, w_ref[...], preferred_element_type=jnp.float32)
        + jnp.dot(s8_ref[0], wf_ref[...], preferred_element_type=jnp.float32))


def _simtopk_body(s_ref, tn_ref, vals_ref, idx_ref):
    b = pl.program_id(0)
    s = s_ref[0]
    sn = s / (jnp.sqrt(jnp.sum(s * s, axis=-1, keepdims=True)) + 1e-8)
    tn = tn_ref[0]
    sim = lax.dot_general(sn, tn, (((1,), (1,)), ((), ())),
                          preferred_element_type=jnp.float32) / TEMPERATURE
    iota = lax.broadcasted_iota(jnp.int32, sim.shape, 1)
    kiota = lax.broadcasted_iota(jnp.int32, (BLK_SIM, TOPK), 1)
    vacc = jnp.zeros((BLK_SIM, TOPK), jnp.float32)
    iacc = jnp.zeros((BLK_SIM, TOPK), jnp.int32)
    for k in range(TOPK):
        m = jnp.max(sim, axis=1)
        mi = jnp.min(jnp.where(sim == m[:, None], iota, N), axis=1)
        vacc = jnp.where(kiota == k, m[:, None], vacc)
        iacc = jnp.where(kiota == k, mi[:, None], iacc)
        if k < TOPK - 1:
            sim = jnp.where(iota == mi[:, None], -1e30, sim)
    vals_ref[0] = vacc
    idx_ref[0] = iacc + b * N


def _refine_body(g_ref, v_ref, sd_ref, ss_ref,
                 w1s_ref, wsf_ref, wv_ref, wd_ref, b1_ref,
                 w2_ref, b2_ref, wlo_ref, blo_ref,
                 exp_ref, disp_ref, probs_ref, conf_ref, marg_ref, ent_ref):
    gall = g_ref[0]
    g = gall[..., 0:H]
    v3 = v_ref[0]
    sd = sd_ref[0]
    ss = ss_ref[0]

    cand_canon = gall[..., H:H + 3]
    cand_pos = gall[..., H + 3:H + 6]
    cm3 = gall[..., H + 6:H + 7]
    delta = cand_canon - ss[:, None, 0:3]
    dist = jnp.sqrt(jnp.sum(delta * delta, axis=-1, keepdims=True))

    srcp = (jnp.dot(sd, w1s_ref[...], preferred_element_type=jnp.float32)
            + jnp.dot(ss, wsf_ref[...], preferred_element_type=jnp.float32)
            + b1_ref[...])
    h = jax.nn.gelu(g + srcp[:, None, :] + v3 * wv_ref[...][None]
                    + dist * wd_ref[...][None])
    h = jax.nn.gelu(
        lax.dot_general(h, w2_ref[...], (((2,), (0,)), ((), ())),
                        preferred_element_type=jnp.float32) + b2_ref[...][None])
    out4 = lax.dot_general(h, wlo_ref[...], (((2,), (0,)), ((), ())),
                           preferred_element_type=jnp.float32) + blo_ref[...][None]

    logits = v3[..., 0] + out4[..., 0]
    offset = jnp.tanh(out4[..., 1:4]) * OFFSET_SCALE
    m = jnp.max(logits, axis=-1, keepdims=True)
    e = jnp.exp(logits - m)
    p = e / jnp.sum(e, axis=-1, keepdims=True)

    refined = cand_pos + offset
    expected = jnp.sum(p[..., None] * refined, axis=1)
    disp = expected - ss[:, 3:6]

    kiota = lax.broadcasted_iota(jnp.int32, p.shape, 1)
    top1 = jnp.max(p, axis=-1, keepdims=True)
    idx1 = jnp.min(jnp.where(p == top1, kiota, TOPK), axis=-1, keepdims=True)
    masked = jnp.where(kiota == idx1, -1.0, p)
    top2 = jnp.max(masked, axis=-1, keepdims=True)
    margin = top1 - top2
    btm = jnp.sum(jnp.where(kiota == idx1, cm3[..., 0], 0.0), axis=-1,
                  keepdims=True)
    conf = top1 * jnp.sqrt(jnp.maximum(ss[:, 7:8] * btm, 1e-6))
    ent = -jnp.sum(p * jnp.log(p + 1e-12), axis=-1, keepdims=True)

    exp_ref[0] = expected
    disp_ref[0] = jnp.nan_to_num(disp, nan=0.0, posinf=0.0, neginf=0.0)
    probs_ref[0] = jnp.nan_to_num(p, nan=0.0, posinf=0.0, neginf=0.0)
    conf_ref[0] = conf
    marg_ref[0] = jnp.nan_to_num(margin, nan=0.0, posinf=0.0, neginf=0.0)
    ent_ref[0] = jnp.nan_to_num(ent, nan=0.0, posinf=0.0, neginf=0.0)


def _gather_sc(tab, idx2):
    rows = idx2.shape[0] * idx2.shape[1]
    info = plsc.get_sparse_core_info()
    nw = info.num_cores * info.num_subcores
    rows_w = rows // nw
    n_chunks = rows_w // GCHUNK
    idx_rows_w = rows_w // idx2.shape[1]
    mesh = plsc.VectorSubcoreMesh(core_axis_name="c", subcore_axis_name="s")

    @functools.partial(
        pl.kernel, mesh=mesh,
        out_type=jax.ShapeDtypeStruct((rows, TABW), jnp.float32),
        scratch_types=[pltpu.VMEM((idx_rows_w, 128), jnp.int32),
                       pltpu.VMEM((GCHUNK, TABW), jnp.float32),
                       pltpu.SemaphoreType.DMA],
    )
    def gk(tab_hbm, idx_hbm, out, idx_v, buf, sem):
        wid = lax.axis_index("s") * info.num_cores + lax.axis_index("c")
        pltpu.sync_copy(idx_hbm.at[pl.ds(wid * idx_rows_w, idx_rows_w)], idx_v)

        def body(c, carry):
            pltpu.async_copy(tab_hbm.at[idx_v.at[c]], buf, sem).wait()
            base = wid * rows_w + c * GCHUNK
            pltpu.sync_copy(buf, out.at[pl.ds(base, GCHUNK)])
            return carry

        lax.fori_loop(0, n_chunks, body, 0)

    return gk(tab, idx2)


def kernel(src_descriptor, tgt_descriptor, src_canonical, tgt_canonical,
           src_positions, tgt_positions, src_uncertainty, tgt_uncertainty,
           src_matchability, tgt_matchability,
           W1, b1, W2, b2, Wl, bl, Wo, bo):
    f32 = jnp.float32

    tgt_small = jnp.concatenate(
        [tgt_canonical, tgt_positions, tgt_uncertainty[..., None],
         tgt_matchability[..., None]], axis=-1)
    w_fold = jnp.concatenate(
        [W1[2 * C + 1:2 * C + 7], W1[2 * C + 9:2 * C + 10],
         W1[2 * C + 11:2 * C + 12]], axis=0)
    tn, tgt_proj = pl.pallas_call(
        _norm_proj_body,
        grid=(B,),
        in_specs=[pl.BlockSpec((1, N, C), lambda b: (b, 0, 0)),
                  pl.BlockSpec((C, H), lambda b: (0, 0)),
                  pl.BlockSpec((1, N, 8), lambda b: (b, 0, 0)),
                  pl.BlockSpec((8, H), lambda b: (0, 0))],
        out_specs=[pl.BlockSpec((1, N, C), lambda b: (b, 0, 0)),
                   pl.BlockSpec((1, N, H), lambda b: (b, 0, 0))],
        out_shape=[jax.ShapeDtypeStruct((B, N, C), f32),
                   jax.ShapeDtypeStruct((B, N, H), f32)],
    )(tgt_descriptor, W1[C:2 * C], tgt_small, w_fold)

    vals, idxf = pl.pallas_call(
        _simtopk_body,
        grid=(B, N // BLK_SIM),
        in_specs=[pl.BlockSpec((1, BLK_SIM, C), lambda b, i: (b, i, 0)),
                  pl.BlockSpec((1, N, C), lambda b, i: (b, 0, 0))],
        out_specs=[pl.BlockSpec((1, BLK_SIM, TOPK), lambda b, i: (b, i, 0)),
                   pl.BlockSpec((1, BLK_SIM, TOPK), lambda b, i: (b, i, 0))],
        out_shape=[jax.ShapeDtypeStruct((B, N, TOPK), f32),
                   jax.ShapeDtypeStruct((B, N, TOPK), jnp.int32)],
    )(src_descriptor, tn)

    tab = jnp.concatenate(
        [tgt_proj, tgt_canonical, tgt_positions, tgt_matchability[..., None],
         jnp.zeros((B, N, TABW - H - 7), f32)], axis=-1)
    rows = B * N * TOPK
    ga = _gather_sc(tab.reshape(B * N, TABW),
                    idxf.reshape(rows // 128, 128))
    g4 = ga.reshape(B, N, TOPK, TABW)

    src_small = jnp.concatenate(
        [src_canonical, src_positions, src_uncertainty[..., None],
         src_matchability[..., None]], axis=-1)
    wlo = jnp.concatenate([Wl, Wo], axis=1)
    blo = jnp.concatenate([bl, bo])[None]
    w_sfold = jnp.concatenate(
        [-W1[2 * C + 1:2 * C + 7], W1[2 * C + 8:2 * C + 9],
         W1[2 * C + 10:2 * C + 11]], axis=0)

    outs = pl.pallas_call(
        _refine_body,
        grid=(B, N // BLK_REF),
        in_specs=[
            pl.BlockSpec((1, BLK_REF, TOPK, TABW), lambda b, i: (b, i, 0, 0)),
            pl.BlockSpec((1, BLK_REF, TOPK, 1), lambda b, i: (b, i, 0, 0)),
            pl.BlockSpec((1, BLK_REF, C), lambda b, i: (b, i, 0)),
            pl.BlockSpec((1, BLK_REF, 8), lambda b, i: (b, i, 0)),
            pl.BlockSpec((C, H), lambda b, i: (0, 0)),
            pl.BlockSpec((8, H), lambda b, i: (0, 0)),
            pl.BlockSpec((1, H), lambda b, i: (0, 0)),
            pl.BlockSpec((1, H), lambda b, i: (0, 0)),
            pl.BlockSpec((1, H), lambda b, i: (0, 0)),
            pl.BlockSpec((H, H), lambda b, i: (0, 0)),
            pl.BlockSpec((1, H), lambda b, i: (0, 0)),
            pl.BlockSpec((H, 4), lambda b, i: (0, 0)),
            pl.BlockSpec((1, 4), lambda b, i: (0, 0)),
        ],
        out_specs=[
            pl.BlockSpec((1, BLK_REF, 3), lambda b, i: (b, i, 0)),
            pl.BlockSpec((1, BLK_REF, 3), lambda b, i: (b, i, 0)),
            pl.BlockSpec((1, BLK_REF, TOPK), lambda b, i: (b, i, 0)),
            pl.BlockSpec((1, BLK_REF, 1), lambda b, i: (b, i, 0)),
            pl.BlockSpec((1, BLK_REF, 1), lambda b, i: (b, i, 0)),
            pl.BlockSpec((1, BLK_REF, 1), lambda b, i: (b, i, 0)),
        ],
        out_shape=[
            jax.ShapeDtypeStruct((B, N, 3), f32),
            jax.ShapeDtypeStruct((B, N, 3), f32),
            jax.ShapeDtypeStruct((B, N, TOPK), f32),
            jax.ShapeDtypeStruct((B, N, 1), f32),
            jax.ShapeDtypeStruct((B, N, 1), f32),
            jax.ShapeDtypeStruct((B, N, 1), f32),
        ],
    )(g4, vals[..., None], src_descriptor, src_small,
      W1[0:C], w_sfold, W1[2 * C:2 * C + 1], W1[2 * C + 7:2 * C + 8],
      b1[None], W2, b2[None], wlo, blo)

    expected, disp, probs, conf, margin, ent = outs
    raw_disp = jnp.transpose(disp, (0, 2, 1)).reshape(B, 3, *SPATIAL)
    confidence = conf.reshape(B, 1, *SPATIAL)
    margin = margin.reshape(B, 1, *SPATIAL)
    entropy = ent.reshape(B, 1, *SPATIAL)
    return (expected, raw_disp, probs, confidence, margin, entropy)
